# baseline (speedup 1.0000x reference)
import functools

import jax
import jax.numpy as jnp
import numpy as np
from jax import lax
from jax.experimental import pallas as pl
from jax.experimental.pallas import tpu as pltpu
from jax.experimental.pallas import tpu_sc as plsc

_B = 2
_IN_C = 8
_OUT_C = 16
_H = 224
_W = 224
_WP = 240
_HP = _H + 2
_ROWS = 14
_BLK = _ROWS + 2
_NV = _W // 16
_NGATES = 128 + 64 + 32 + 16

_OP_BASIS = np.array(
    [
        [0, 0, 0, 0, 0, 0, 0, 0, 1, 1, 1, 1, 1, 1, 1, 1],
        [0, 0, 1, 1, 0, 0, 1, 1, -1, -1, 0, 0, -1, -1, 0, 0],
        [0, 0, 0, 0, 1, 1, 1, 1, -1, -1, -1, -1, 0, 0, 0, 0],
        [0, 1, -1, 0, -1, 0, -2, -1, 1, 2, 0, 1, 0, 1, -1, 0],
    ],
    dtype=np.float32,
)


def _tree_body(xp_hbm, codes_hbm, wt_hbm, out_hbm,
               xb, codes_v, wt_v, coef_v, o0, o1, o2, o3, sem0, sem1, sem_in):
    cid = lax.axis_index("c")
    sid = lax.axis_index("s")
    wid = cid * 16 + sid
    b = wid // 16
    r0 = (wid % 16) * _ROWS

    in_copies = [
        pltpu.async_copy(
            xp_hbm.at[b, pl.ds(c * _HP + r0, _BLK), :],
            xb.at[pl.ds(c * _BLK, _BLK), :],
            sem_in,
        )
        for c in range(_IN_C)
    ]
    pltpu.sync_copy(codes_hbm, codes_v)
    pltpu.sync_copy(wt_hbm, wt_v)

    @plsc.parallel_loop(0, _NGATES // 16)
    def coef_group(g):
        vs = [wt_v[k, pl.ds(16 * g, 16)] for k in range(16)]
        m = vs[0]
        for k in range(1, 16):
            m = jnp.maximum(m, vs[k])
        es = [jnp.exp(v - m) for v in vs]
        s = es[0]
        for k in range(1, 16):
            s = s + es[k]
        rs = 1.0 / s
        for ci in range(4):
            acc = None
            for k in range(16):
                cw = float(_OP_BASIS[ci, k])
                if cw == 0.0:
                    continue
                term = es[k] if cw == 1.0 else es[k] * cw
                acc = term if acc is None else acc + term
            accv = acc * rs
            for i in range(16):
                coef_v[16 * g + i, ci, :] = jnp.full((16,), accv[i], jnp.float32)

    for cp in in_copies:
        cp.wait()

    def row_body(rr, carry):
        @plsc.parallel_loop(0, 128)
        def l0_body(j):
            cvec = codes_v[j, :]
            k0 = cvec[0]
            k1 = cvec[1]
            ra = (k0 >> 2) + rr
            rb = (k1 >> 2) + rr
            x0 = k0 & 3
            x1 = k1 & 3
            av = coef_v[j, 0, :]
            bv = coef_v[j, 1, :]
            gv = coef_v[j, 2, :]
            dv = coef_v[j, 3, :]
            @plsc.parallel_loop(0, _NV)
            def vloop(v):
                a = xb[ra, pl.ds(x0 + 16 * v, 16)]
                bb = xb[rb, pl.ds(x1 + 16 * v, 16)]
                o0[j, pl.ds(16 * v, 16)] = a * (dv * bb + bv) + (gv * bb + av)

        def run_layer(src, dst, off, n):
            @plsc.parallel_loop(0, n)
            def body(j):
                av = coef_v[off + j, 0, :]
                bv = coef_v[off + j, 1, :]
                gv = coef_v[off + j, 2, :]
                dv = coef_v[off + j, 3, :]
                @plsc.parallel_loop(0, _NV)
                def vloop(v):
                    a = src[2 * j, pl.ds(16 * v, 16)]
                    bb = src[2 * j + 1, pl.ds(16 * v, 16)]
                    dst[j, pl.ds(16 * v, 16)] = a * (dv * bb + bv) + (gv * bb + av)

        run_layer(o0, o1, 128, 64)
        run_layer(o1, o2, 192, 32)
        return carry

    def row_pair(h, carry):
        for par, semp in ((0, sem0), (1, sem1)):
            rr = 2 * h + par
            row_body(rr, 0)

            @pl.when(h > 0)
            def _():
                pltpu.make_async_copy(o3.at[par], out_hbm.at[b, :, 0, :],
                                      semp).wait()

            @plsc.parallel_loop(0, 16)
            def l3_body(j):
                av = coef_v[224 + j, 0, :]
                bv = coef_v[224 + j, 1, :]
                gv = coef_v[224 + j, 2, :]
                dv = coef_v[224 + j, 3, :]
                for v in range(_NV):
                    a = o2[2 * j, pl.ds(16 * v, 16)]
                    bb = o2[2 * j + 1, pl.ds(16 * v, 16)]
                    o3[par, j, pl.ds(16 * v, 16)] = (
                        a * (dv * bb + bv) + (gv * bb + av))

            pltpu.async_copy(o3.at[par], out_hbm.at[b, :, r0 + rr, :], semp)
        return carry

    lax.fori_loop(0, _ROWS // 2, row_pair, 0)
    pltpu.make_async_copy(o3.at[0], out_hbm.at[b, :, 0, :], sem0).wait()
    pltpu.make_async_copy(o3.at[1], out_hbm.at[b, :, 0, :], sem1).wait()


_tree_call = functools.partial(
    pl.kernel,
    out_type=jax.ShapeDtypeStruct((_B, _OUT_C, _H, _W), jnp.float32),
    mesh=plsc.VectorSubcoreMesh(
        core_axis_name="c", subcore_axis_name="s", num_cores=2, num_subcores=16
    ),
    compiler_params=pltpu.CompilerParams(use_tc_tiling_on_sc=False),
    scratch_types=[
        pltpu.VMEM((_IN_C * _BLK, _WP), jnp.float32),
        pltpu.VMEM((128, 16), jnp.int32),
        pltpu.VMEM((16, _NGATES), jnp.float32),
        pltpu.VMEM((_NGATES, 4, 16), jnp.float32),
        pltpu.VMEM((128, _W), jnp.float32),
        pltpu.VMEM((64, _W), jnp.float32),
        pltpu.VMEM((32, _W), jnp.float32),
        pltpu.VMEM((2, 16, _W), jnp.float32),
        pltpu.SemaphoreType.DMA,
        pltpu.SemaphoreType.DMA,
        pltpu.SemaphoreType.DMA,
    ],
)(_tree_body)


def kernel(x, indices, in_idx, w0, w1, w2, w3):
    xp = jnp.pad(x, ((0, 0), (0, 0), (1, 1), (1, _WP - _W - 1)))
    xp = xp.reshape(_B, _IN_C * _HP, _WP)
    kf = indices.reshape(-1).astype(jnp.int32)
    irow = jnp.repeat(jnp.arange(_OUT_C, dtype=jnp.int32), _OUT_C)
    ch = in_idx[irow, kf // 9].astype(jnp.int32)
    pos = kf % 9
    codes = (ch * _BLK + pos // 3) * 4 + (pos % 3)
    codes = jnp.pad(codes.reshape(128, 2), ((0, 0), (0, 14)))
    w_all = jnp.concatenate([w0, w1, w2, w3], axis=0)
    wt = w_all.T
    return _tree_call(xp, codes.astype(jnp.int32), wt)


# speedup vs baseline: 2.2428x; 2.2428x over previous
import functools

import jax
import jax.numpy as jnp
import numpy as np
from jax import lax
from jax.experimental import pallas as pl
from jax.experimental.pallas import tpu as pltpu
from jax.experimental.pallas import tpu_sc as plsc

_B = 2
_IN_C = 8
_OUT_C = 16
_H = 224
_W = 224
_WP = 240
_HP = _H + 2
_ROWS = 14
_BLK = _ROWS + 2
_NV = _W // 16
_NGATES = 128 + 64 + 32 + 16

_OP_BASIS = np.array(
    [
        [0, 0, 0, 0, 0, 0, 0, 0, 1, 1, 1, 1, 1, 1, 1, 1],
        [0, 0, 1, 1, 0, 0, 1, 1, -1, -1, 0, 0, -1, -1, 0, 0],
        [0, 0, 0, 0, 1, 1, 1, 1, -1, -1, -1, -1, 0, 0, 0, 0],
        [0, 1, -1, 0, -1, 0, -2, -1, 1, 2, 0, 1, 0, 1, -1, 0],
    ],
    dtype=np.float32,
)


def _tree_body(xp_hbm, codes_hbm, wt_hbm, out_hbm,
               xb, codes_v, wt_v, coef_v, o0, o1, o2, o3, sem0, sem1, sem_in):
    cid = lax.axis_index("c")
    sid = lax.axis_index("s")
    wid = cid * 16 + sid
    b = wid // 16
    r0 = (wid % 16) * _ROWS

    in_copies = [
        pltpu.async_copy(
            xp_hbm.at[b, pl.ds(c * _HP + r0, _BLK), :],
            xb.at[pl.ds(c * _BLK, _BLK), :],
            sem_in,
        )
        for c in range(_IN_C)
    ]
    pltpu.sync_copy(codes_hbm, codes_v)
    pltpu.sync_copy(wt_hbm, wt_v)

    @plsc.parallel_loop(0, _NGATES // 16)
    def coef_group(g):
        vs = [wt_v[k, pl.ds(16 * g, 16)] for k in range(16)]
        m = vs[0]
        for k in range(1, 16):
            m = jnp.maximum(m, vs[k])
        es = [jnp.exp(v - m) for v in vs]
        s = es[0]
        for k in range(1, 16):
            s = s + es[k]
        rs = 1.0 / s
        for ci in range(4):
            acc = None
            for k in range(16):
                cw = float(_OP_BASIS[ci, k])
                if cw == 0.0:
                    continue
                term = es[k] if cw == 1.0 else es[k] * cw
                acc = term if acc is None else acc + term
            accv = acc * rs
            for i in range(16):
                coef_v[16 * g + i, ci, :] = jnp.full((16,), accv[i], jnp.float32)

    for cp in in_copies:
        cp.wait()

    def row_body(rr, carry):
        @plsc.parallel_loop(0, 128)
        def l0_body(j):
            cvec = codes_v[j, :]
            k0 = cvec[0]
            k1 = cvec[1]
            ra = (k0 >> 2) + rr
            rb = (k1 >> 2) + rr
            x0 = k0 & 3
            x1 = k1 & 3
            av = coef_v[j, 0, :]
            bv = coef_v[j, 1, :]
            gv = coef_v[j, 2, :]
            dv = coef_v[j, 3, :]
            for v in range(_NV):
                a = xb[ra, pl.ds(x0 + 16 * v, 16)]
                bb = xb[rb, pl.ds(x1 + 16 * v, 16)]
                o0[j, pl.ds(16 * v, 16)] = a * (dv * bb + bv) + (gv * bb + av)

        def run_layer(src, dst, off, n):
            @plsc.parallel_loop(0, n)
            def body(j):
                av = coef_v[off + j, 0, :]
                bv = coef_v[off + j, 1, :]
                gv = coef_v[off + j, 2, :]
                dv = coef_v[off + j, 3, :]
                for v in range(_NV):
                    a = src[2 * j, pl.ds(16 * v, 16)]
                    bb = src[2 * j + 1, pl.ds(16 * v, 16)]
                    dst[j, pl.ds(16 * v, 16)] = a * (dv * bb + bv) + (gv * bb + av)

        run_layer(o0, o1, 128, 64)
        run_layer(o1, o2, 192, 32)
        return carry

    def row_pair(h, carry):
        for par, semp in ((0, sem0), (1, sem1)):
            rr = 2 * h + par
            row_body(rr, 0)

            @pl.when(h > 0)
            def _():
                pltpu.make_async_copy(o3.at[par], out_hbm.at[b, :, 0, :],
                                      semp).wait()

            @plsc.parallel_loop(0, 16)
            def l3_body(j):
                av = coef_v[224 + j, 0, :]
                bv = coef_v[224 + j, 1, :]
                gv = coef_v[224 + j, 2, :]
                dv = coef_v[224 + j, 3, :]
                for v in range(_NV):
                    a = o2[2 * j, pl.ds(16 * v, 16)]
                    bb = o2[2 * j + 1, pl.ds(16 * v, 16)]
                    o3[par, j, pl.ds(16 * v, 16)] = (
                        a * (dv * bb + bv) + (gv * bb + av))

            pltpu.async_copy(o3.at[par], out_hbm.at[b, :, r0 + rr, :], semp)
        return carry

    lax.fori_loop(0, _ROWS // 2, row_pair, 0)
    pltpu.make_async_copy(o3.at[0], out_hbm.at[b, :, 0, :], sem0).wait()
    pltpu.make_async_copy(o3.at[1], out_hbm.at[b, :, 0, :], sem1).wait()


_tree_call = functools.partial(
    pl.kernel,
    out_type=jax.ShapeDtypeStruct((_B, _OUT_C, _H, _W), jnp.float32),
    mesh=plsc.VectorSubcoreMesh(
        core_axis_name="c", subcore_axis_name="s", num_cores=2, num_subcores=16
    ),
    compiler_params=pltpu.CompilerParams(use_tc_tiling_on_sc=False),
    scratch_types=[
        pltpu.VMEM((_IN_C * _BLK, _WP), jnp.float32),
        pltpu.VMEM((128, 16), jnp.int32),
        pltpu.VMEM((16, _NGATES), jnp.float32),
        pltpu.VMEM((_NGATES, 4, 16), jnp.float32),
        pltpu.VMEM((128, _W), jnp.float32),
        pltpu.VMEM((64, _W), jnp.float32),
        pltpu.VMEM((32, _W), jnp.float32),
        pltpu.VMEM((2, 16, _W), jnp.float32),
        pltpu.SemaphoreType.DMA,
        pltpu.SemaphoreType.DMA,
        pltpu.SemaphoreType.DMA,
    ],
)(_tree_body)


def kernel(x, indices, in_idx, w0, w1, w2, w3):
    xp = jnp.pad(x, ((0, 0), (0, 0), (1, 1), (1, _WP - _W - 1)))
    xp = xp.reshape(_B, _IN_C * _HP, _WP)
    kf = indices.reshape(-1).astype(jnp.int32)
    irow = jnp.repeat(jnp.arange(_OUT_C, dtype=jnp.int32), _OUT_C)
    ch = in_idx[irow, kf // 9].astype(jnp.int32)
    pos = kf % 9
    codes = (ch * _BLK + pos // 3) * 4 + (pos % 3)
    codes = jnp.pad(codes.reshape(128, 2), ((0, 0), (0, 14)))
    w_all = jnp.concatenate([w0, w1, w2, w3], axis=0)
    wt = w_all.T
    return _tree_call(xp, codes.astype(jnp.int32), wt)


# speedup vs baseline: 2.2555x; 1.0056x over previous
import functools

import jax
import jax.numpy as jnp
import numpy as np
from jax import lax
from jax.experimental import pallas as pl
from jax.experimental.pallas import tpu as pltpu
from jax.experimental.pallas import tpu_sc as plsc

_B = 2
_IN_C = 8
_OUT_C = 16
_H = 224
_W = 224
_WP = 240
_HP = _H + 2
_ROWS = 14
_BLK = _ROWS + 2
_NV = _W // 16
_NGATES = 128 + 64 + 32 + 16

_OP_BASIS = np.array(
    [
        [0, 0, 0, 0, 0, 0, 0, 0, 1, 1, 1, 1, 1, 1, 1, 1],
        [0, 0, 1, 1, 0, 0, 1, 1, -1, -1, 0, 0, -1, -1, 0, 0],
        [0, 0, 0, 0, 1, 1, 1, 1, -1, -1, -1, -1, 0, 0, 0, 0],
        [0, 1, -1, 0, -1, 0, -2, -1, 1, 2, 0, 1, 0, 1, -1, 0],
    ],
    dtype=np.float32,
)


def _tree_body(xp_hbm, codes_hbm, wt_hbm, out_hbm,
               xb, codes_v, wt_v, coef_v, o0, o1, o3, sem0, sem1, sem_in):
    cid = lax.axis_index("c")
    sid = lax.axis_index("s")
    wid = cid * 16 + sid
    b = wid // 16
    r0 = (wid % 16) * _ROWS

    in_copies = [
        pltpu.async_copy(
            xp_hbm.at[b, pl.ds(c * _HP + r0, _BLK), :],
            xb.at[pl.ds(c * _BLK, _BLK), :],
            sem_in,
        )
        for c in range(_IN_C)
    ]
    pltpu.sync_copy(codes_hbm, codes_v)
    pltpu.sync_copy(wt_hbm, wt_v)

    @plsc.parallel_loop(0, _NGATES // 16)
    def coef_group(g):
        vs = [wt_v[k, pl.ds(16 * g, 16)] for k in range(16)]
        m = vs[0]
        for k in range(1, 16):
            m = jnp.maximum(m, vs[k])
        es = [jnp.exp(v - m) for v in vs]
        s = es[0]
        for k in range(1, 16):
            s = s + es[k]
        rs = 1.0 / s
        for ci in range(4):
            acc = None
            for k in range(16):
                cw = float(_OP_BASIS[ci, k])
                if cw == 0.0:
                    continue
                term = es[k] if cw == 1.0 else es[k] * cw
                acc = term if acc is None else acc + term
            accv = acc * rs
            for i in range(16):
                coef_v[16 * g + i, ci, :] = jnp.full((16,), accv[i], jnp.float32)

    for cp in in_copies:
        cp.wait()

    def row_body(rr, carry):
        @plsc.parallel_loop(0, 128)
        def l0_body(j):
            cvec = codes_v[j, :]
            k0 = cvec[0]
            k1 = cvec[1]
            ra = (k0 >> 2) + rr
            rb = (k1 >> 2) + rr
            x0 = k0 & 3
            x1 = k1 & 3
            av = coef_v[j, 0, :]
            bv = coef_v[j, 1, :]
            gv = coef_v[j, 2, :]
            dv = coef_v[j, 3, :]
            for v in range(_NV):
                a = xb[ra, pl.ds(x0 + 16 * v, 16)]
                bb = xb[rb, pl.ds(x1 + 16 * v, 16)]
                o0[j, pl.ds(16 * v, 16)] = a * (dv * bb + bv) + (gv * bb + av)

        def run_layer(src, dst, off, n):
            @plsc.parallel_loop(0, n)
            def body(j):
                av = coef_v[off + j, 0, :]
                bv = coef_v[off + j, 1, :]
                gv = coef_v[off + j, 2, :]
                dv = coef_v[off + j, 3, :]
                for v in range(_NV):
                    a = src[2 * j, pl.ds(16 * v, 16)]
                    bb = src[2 * j + 1, pl.ds(16 * v, 16)]
                    dst[j, pl.ds(16 * v, 16)] = a * (dv * bb + bv) + (gv * bb + av)

        run_layer(o0, o1, 128, 64)
        return carry

    def gate(a, b, cf):
        av, bv, gv, dv = cf
        return a * (dv * b + bv) + (gv * b + av)

    def row_pair(h, carry):
        for par, semp in ((0, sem0), (1, sem1)):
            rr = 2 * h + par
            row_body(rr, 0)

            @pl.when(h > 0)
            def _():
                pltpu.make_async_copy(o3.at[par], out_hbm.at[b, :, 0, :],
                                      semp).wait()

            @plsc.parallel_loop(0, 16)
            def l23_body(j):
                c2 = [tuple(coef_v[192 + 2 * j + t, ci, :] for ci in range(4))
                      for t in range(2)]
                c3 = tuple(coef_v[224 + j, ci, :] for ci in range(4))
                for v in range(_NV):
                    s = pl.ds(16 * v, 16)
                    y = [o1[4 * j + t, s] for t in range(4)]
                    z0 = gate(y[0], y[1], c2[0])
                    z1 = gate(y[2], y[3], c2[1])
                    o3[par, j, s] = gate(z0, z1, c3)

            pltpu.async_copy(o3.at[par], out_hbm.at[b, :, r0 + rr, :], semp)
        return carry

    lax.fori_loop(0, _ROWS // 2, row_pair, 0)
    pltpu.make_async_copy(o3.at[0], out_hbm.at[b, :, 0, :], sem0).wait()
    pltpu.make_async_copy(o3.at[1], out_hbm.at[b, :, 0, :], sem1).wait()


_tree_call = functools.partial(
    pl.kernel,
    out_type=jax.ShapeDtypeStruct((_B, _OUT_C, _H, _W), jnp.float32),
    mesh=plsc.VectorSubcoreMesh(
        core_axis_name="c", subcore_axis_name="s", num_cores=2, num_subcores=16
    ),
    compiler_params=pltpu.CompilerParams(use_tc_tiling_on_sc=False),
    scratch_types=[
        pltpu.VMEM((_IN_C * _BLK, _WP), jnp.float32),
        pltpu.VMEM((128, 16), jnp.int32),
        pltpu.VMEM((16, _NGATES), jnp.float32),
        pltpu.VMEM((_NGATES, 4, 16), jnp.float32),
        pltpu.VMEM((128, _W), jnp.float32),
        pltpu.VMEM((64, _W), jnp.float32),
        pltpu.VMEM((2, 16, _W), jnp.float32),
        pltpu.SemaphoreType.DMA,
        pltpu.SemaphoreType.DMA,
        pltpu.SemaphoreType.DMA,
    ],
)(_tree_body)


def kernel(x, indices, in_idx, w0, w1, w2, w3):
    xp = jnp.pad(x, ((0, 0), (0, 0), (1, 1), (1, _WP - _W - 1)))
    xp = xp.reshape(_B, _IN_C * _HP, _WP)
    kf = indices.reshape(-1).astype(jnp.int32)
    irow = jnp.repeat(jnp.arange(_OUT_C, dtype=jnp.int32), _OUT_C)
    ch = in_idx[irow, kf // 9].astype(jnp.int32)
    pos = kf % 9
    codes = (ch * _BLK + pos // 3) * 4 + (pos % 3)
    codes = jnp.pad(codes.reshape(128, 2), ((0, 0), (0, 14)))
    w_all = jnp.concatenate([w0, w1, w2, w3], axis=0)
    wt = w_all.T
    return _tree_call(xp, codes.astype(jnp.int32), wt)
